# Initial kernel scaffold; baseline (speedup 1.0000x reference)
import jax
import jax.numpy as jnp
from jax.experimental import pallas as pl


def kernel(inp_embed, out_embed, inp, out, neg):
    raise NotImplementedError("write your pallas kernel here")


# speedup vs baseline: 1.3476x; 1.3476x over previous
import functools

import jax
import jax.numpy as jnp
from jax import lax
from jax.experimental import pallas as pl
from jax.experimental.pallas import tpu as pltpu
from jax.experimental.pallas import tpu_sc as plsc

B = 16384
V = 1000000
D = 32
CTX = 20
NEG = 20

NC = 2
NS = 16
NW = NC * NS
RPW = B // NW
CH = 128
NCHUNK = RPW // CH
IPC = CH * CTX
KROWS = IPC // 128
IDXROWS_PW = RPW * CTX // 128


def _sc_body(inp_embed, out_embed, inp2d, out2d, neg2d,
             pos_hbm, negT_hbm,
             idxa, idxb, idxo, rows_v, out_rows, ctx_v, pos_v, negT_v,
             sem, sem2):
    wid = lax.axis_index("s") * NC + lax.axis_index("c")
    iota = lax.iota(jnp.int32, 16)

    def chunk_body(c, carry):
        ioff = (wid * RPW + c * CH) * CTX
        pltpu.sync_copy(inp2d.at[pl.ds(ioff, IPC)], idxa)
        handles = []
        for k in range(KROWS):
            handles.append(pltpu.async_copy(
                inp_embed.at[idxa.at[pl.ds(k * 128, 128)]],
                rows_v.at[pl.ds(k * 128, 128)], sem))
        pltpu.sync_copy(out2d.at[pl.ds(wid * RPW + c * CH, CH)], idxo)
        oh = pltpu.async_copy(out_embed.at[idxo], out_rows, sem2)
        for h in handles:
            h.wait()

        def ctx_sum_body(i, cc):
            m0 = i * CTX
            a0 = rows_v[m0, pl.ds(0, 16)]
            a1 = rows_v[m0, pl.ds(16, 16)]
            for k in range(1, CTX):
                a0 = a0 + rows_v[m0 + k, pl.ds(0, 16)]
                a1 = a1 + rows_v[m0 + k, pl.ds(16, 16)]
            ctx_v[i, pl.ds(0, 16)] = a0
            ctx_v[i, pl.ds(16, 16)] = a1
            return cc
        lax.fori_loop(0, CH, ctx_sum_body, 0)

        pltpu.sync_copy(neg2d.at[pl.ds(ioff, IPC)], idxb)
        nh = []
        for k in range(KROWS):
            nh.append(pltpu.async_copy(
                out_embed.at[idxb.at[pl.ds(k * 128, 128)]],
                rows_v.at[pl.ds(k * 128, 128)], sem))
        oh.wait()
        for h in nh:
            h.wait()

        def group_body(g, cc):
            rows16 = g * 16 + iota
            negrows = [rows16 * NEG + j for j in range(NEG)]

            def d_body(dcol, acc):
                col = jnp.full((16,), dcol, jnp.int32)
                cd = plsc.load_gather(ctx_v, [rows16, col])
                od = plsc.load_gather(out_rows, [rows16, col])
                pos = acc[0] + cd * od
                new = tuple(
                    acc[1 + j] + plsc.load_gather(rows_v, [negrows[j], col]) * cd
                    for j in range(NEG))
                return (pos,) + new

            init = tuple(jnp.zeros((16,), jnp.float32) for _ in range(NEG + 1))
            res = lax.fori_loop(0, D, d_body, init)
            off = c * CH + g * 16
            pos_v[pl.ds(off, 16)] = res[0]
            for j in range(NEG):
                negT_v[j, pl.ds(off, 16)] = res[1 + j]
            return cc
        lax.fori_loop(0, CH // 16, group_body, 0)
        return carry

    lax.fori_loop(0, NCHUNK, chunk_body, 0)
    pltpu.sync_copy(pos_v, pos_hbm.at[pl.ds(wid * RPW, RPW)])
    pltpu.sync_copy(negT_v, negT_hbm.at[:, pl.ds(wid * RPW, RPW)])


_sc_call = pl.kernel(
    _sc_body,
    out_type=[jax.ShapeDtypeStruct((B,), jnp.float32),
              jax.ShapeDtypeStruct((NEG, B), jnp.float32)],
    mesh=plsc.VectorSubcoreMesh(core_axis_name="c", subcore_axis_name="s"),
    compiler_params=pltpu.CompilerParams(needs_layout_passes=False,
                                         use_tc_tiling_on_sc=False),
    scratch_types=[
        pltpu.VMEM((IPC,), jnp.int32),
        pltpu.VMEM((IPC,), jnp.int32),
        pltpu.VMEM((CH,), jnp.int32),
        pltpu.VMEM((IPC, D), jnp.float32),
        pltpu.VMEM((CH, D), jnp.float32),
        pltpu.VMEM((CH, D), jnp.float32),
        pltpu.VMEM((RPW,), jnp.float32),
        pltpu.VMEM((NEG, RPW), jnp.float32),
        pltpu.SemaphoreType.DMA,
        pltpu.SemaphoreType.DMA,
    ],
)


def _log_sigmoid(x):
    return jnp.minimum(x, 0.0) - jnp.log(1.0 + jnp.exp(-jnp.abs(x)))


def _tc_body(pos_ref, neg_ref, o_ref):
    pos = pos_ref[...] * (1.0 / CTX)
    neg = neg_ref[...] * (1.0 / CTX)
    t1 = jnp.mean(_log_sigmoid(pos))
    t2 = jnp.sum(_log_sigmoid(-neg)) * (1.0 / B)
    o_ref[0, 0] = -(t1 + t2)


_tc_call = pl.pallas_call(
    _tc_body,
    out_shape=jax.ShapeDtypeStruct((1, 1), jnp.float32),
    out_specs=pl.BlockSpec(memory_space=pltpu.SMEM),
)


def kernel(inp_embed, out_embed, inp, out, neg):
    inp_i = inp.astype(jnp.int32).reshape(B * CTX)
    out_i = out.astype(jnp.int32).reshape(B)
    neg_i = neg.astype(jnp.int32).reshape(B * NEG)
    pos, negT = _sc_call(inp_embed, out_embed, inp_i, out_i, neg_i)
    loss = _tc_call(pos.reshape(128, 128), negT.reshape(NEG * B // 128, 128))
    return loss[0, 0]
